# baseline (speedup 1.0000x reference)
import jax
import jax.numpy as jnp
from jax import lax
from jax.experimental import pallas as pl
from jax.experimental.pallas import tpu as pltpu
from jax.experimental.pallas import tpu_sc as plsc

VOCAB = 1000000
EMBED_DIM = 32
NUM_CORES = 2
NUM_SUBCORES = 16
NW = NUM_CORES * NUM_SUBCORES
CHUNK = 128


def _sc_embed_add(table, xf, cf, n_rows):
    n_chunks = xf.shape[1]
    b_per_w = n_chunks * CHUNK
    mesh = plsc.VectorSubcoreMesh(core_axis_name="c", subcore_axis_name="s")

    def body(table_hbm, x_hbm, c_hbm, out_hbm, xv, cv, buf, sem):
        wid = lax.axis_index("s") * NUM_CORES + lax.axis_index("c")
        pltpu.sync_copy(x_hbm.at[wid], xv)
        pltpu.sync_copy(c_hbm.at[wid], cv)
        row0 = wid * b_per_w

        def chunk(j, carry):
            base = row0 + j * CHUNK
            pltpu.async_copy(table_hbm.at[xv.at[j]], buf, sem).wait()
            pltpu.async_copy(table_hbm.at[cv.at[j]], buf, sem, add=True).wait()
            pltpu.sync_copy(buf, out_hbm.at[pl.ds(base, CHUNK)])
            return carry

        lax.fori_loop(0, n_chunks, chunk, 0)

    run = pl.kernel(
        body,
        out_type=jax.ShapeDtypeStruct((n_rows, EMBED_DIM), jnp.float32),
        mesh=mesh,
        scratch_types=[
            pltpu.VMEM((n_chunks, CHUNK), jnp.int32),
            pltpu.VMEM((n_chunks, CHUNK), jnp.int32),
            pltpu.VMEM((CHUNK, EMBED_DIM), jnp.float32),
            pltpu.SemaphoreType.DMA,
        ],
        compiler_params=pltpu.CompilerParams(use_tc_tiling_on_sc=False),
    )
    return run(table, xf, cf)


def kernel(x, constant, table):
    shape = x.shape
    n_rows = x.size
    xf = x.reshape(NW, n_rows // (NW * CHUNK), CHUNK).astype(jnp.int32)
    cf = constant.reshape(NW, n_rows // (NW * CHUNK), CHUNK).astype(jnp.int32)
    out = _sc_embed_add(table, xf, cf, n_rows)
    return out.reshape(*shape, EMBED_DIM)


# speedup vs baseline: 1.0819x; 1.0819x over previous
import jax
import jax.numpy as jnp
from jax import lax
from jax.experimental import pallas as pl
from jax.experimental.pallas import tpu as pltpu
from jax.experimental.pallas import tpu_sc as plsc

EMBED_DIM = 32
NUM_CORES = 2
NUM_SUBCORES = 16
NW = NUM_CORES * NUM_SUBCORES
CHUNK = 128
K = 5
NBUF = 3
PHASE_ROWS = K * CHUNK


def _sc_embed_add(table, xf, cf, n_rows):
    n_chunks = xf.shape[1]
    b_per_w = n_chunks * CHUNK
    n_phases = n_chunks // K
    mesh = plsc.VectorSubcoreMesh(core_axis_name="c", subcore_axis_name="s")

    def body(table_hbm, x_hbm, c_hbm, out_hbm, xv, cv, bufs, semg, semo):
        wid = lax.axis_index("s") * NUM_CORES + lax.axis_index("c")
        pltpu.sync_copy(x_hbm.at[wid], xv)
        pltpu.sync_copy(c_hbm.at[wid], cv)
        row0 = wid * b_per_w

        def drain_gathers(b):
            pltpu.make_async_copy(
                out_hbm.at[pl.ds(0, PHASE_ROWS)], bufs.at[b], semg.at[b]
            ).wait()

        def drain_out(b):
            pltpu.make_async_copy(
                bufs.at[b], out_hbm.at[pl.ds(0, PHASE_ROWS)], semo.at[b]
            ).wait()

        def step(p, carry):
            b_a = lax.rem(p, NBUF)
            b_b = lax.rem(p + (NBUF - 1), NBUF)
            b_c = lax.rem(p + (NBUF - 2), NBUF)

            @pl.when(jnp.logical_and(p >= 2, p <= n_phases + 1))
            def _():
                drain_gathers(b_c)
                pltpu.async_copy(
                    bufs.at[b_c],
                    out_hbm.at[pl.ds(row0 + (p - 2) * PHASE_ROWS, PHASE_ROWS)],
                    semo.at[b_c],
                )

            @pl.when(jnp.logical_and(p >= 1, p <= n_phases))
            def _():
                drain_gathers(b_b)
                for i in range(K):
                    pltpu.async_copy(
                        table_hbm.at[cv.at[(p - 1) * K + i]],
                        bufs.at[b_b].at[pl.ds(i * CHUNK, CHUNK)],
                        semg.at[b_b],
                        add=True,
                    )

            @pl.when(p <= n_phases - 1)
            def _():
                @pl.when(p >= NBUF)
                def _():
                    drain_out(b_a)

                for i in range(K):
                    pltpu.async_copy(
                        table_hbm.at[xv.at[p * K + i]],
                        bufs.at[b_a].at[pl.ds(i * CHUNK, CHUNK)],
                        semg.at[b_a],
                    )

            return carry

        lax.fori_loop(0, n_phases + 2, step, 0)
        for b in range(NBUF):
            drain_out(b)

    run = pl.kernel(
        body,
        out_type=jax.ShapeDtypeStruct((n_rows, EMBED_DIM), jnp.float32),
        mesh=mesh,
        scratch_types=[
            pltpu.VMEM((n_chunks, CHUNK), jnp.int32),
            pltpu.VMEM((n_chunks, CHUNK), jnp.int32),
            pltpu.VMEM((NBUF, PHASE_ROWS, EMBED_DIM), jnp.float32),
            pltpu.SemaphoreType.DMA((NBUF,)),
            pltpu.SemaphoreType.DMA((NBUF,)),
        ],
        compiler_params=pltpu.CompilerParams(use_tc_tiling_on_sc=False),
    )
    return run(table, xf, cf)


def kernel(x, constant, table):
    shape = x.shape
    n_rows = x.size
    xf = x.reshape(NW, n_rows // (NW * CHUNK), CHUNK).astype(jnp.int32)
    cf = constant.reshape(NW, n_rows // (NW * CHUNK), CHUNK).astype(jnp.int32)
    out = _sc_embed_add(table, xf, cf, n_rows)
    return out.reshape(*shape, EMBED_DIM)


# speedup vs baseline: 1.3263x; 1.2259x over previous
import jax
import jax.numpy as jnp
from jax import lax
from jax.experimental import pallas as pl
from jax.experimental.pallas import tpu as pltpu
from jax.experimental.pallas import tpu_sc as plsc

EMBED_DIM = 32
NUM_CORES = 2
NUM_SUBCORES = 16
NW = NUM_CORES * NUM_SUBCORES
CHUNK = 128
K = 5
NBUF = 3
PHASE_ROWS = K * CHUNK


def _sc_embed_add(table, xf, cf, d0, d1_size):
    n_chunks = xf.shape[1]
    n_phases = n_chunks // K
    mesh = plsc.VectorSubcoreMesh(core_axis_name="c", subcore_axis_name="s")

    def body(table_hbm, x_hbm, c_hbm, out_hbm, xv, cv, bufs, semg, semo):
        wid = lax.axis_index("s") * NUM_CORES + lax.axis_index("c")
        pltpu.sync_copy(x_hbm.at[wid], xv)
        pltpu.sync_copy(c_hbm.at[wid], cv)

        def drain_gathers(b):
            pltpu.make_async_copy(
                table_hbm.at[pl.ds(0, PHASE_ROWS)], bufs.at[b], semg.at[b]
            ).wait()

        def drain_out(b):
            pltpu.make_async_copy(
                bufs.at[b], out_hbm.at[pl.ds(0, PHASE_ROWS), 0], semo.at[b]
            ).wait()

        def step(p, carry):
            b_a = lax.rem(p, NBUF)
            b_b = lax.rem(p + (NBUF - 1), NBUF)
            b_c = lax.rem(p + (NBUF - 2), NBUF)

            @pl.when(jnp.logical_and(p >= 2, p <= n_phases + 1))
            def _():
                drain_gathers(b_c)
                for i in range(K):
                    pair = wid * n_chunks + (p - 2) * K + i
                    d1 = lax.div(pair, NW)
                    j = lax.rem(pair, NW)
                    pltpu.async_copy(
                        bufs.at[b_c].at[pl.ds(i * CHUNK, CHUNK)],
                        out_hbm.at[pl.ds(j * CHUNK, CHUNK), d1],
                        semo.at[b_c],
                    )

            @pl.when(jnp.logical_and(p >= 1, p <= n_phases))
            def _():
                drain_gathers(b_b)
                for i in range(K):
                    pltpu.async_copy(
                        table_hbm.at[cv.at[(p - 1) * K + i]],
                        bufs.at[b_b].at[pl.ds(i * CHUNK, CHUNK)],
                        semg.at[b_b],
                        add=True,
                    )

            @pl.when(p <= n_phases - 1)
            def _():
                @pl.when(p >= NBUF)
                def _():
                    drain_out(b_a)

                for i in range(K):
                    pltpu.async_copy(
                        table_hbm.at[xv.at[p * K + i]],
                        bufs.at[b_a].at[pl.ds(i * CHUNK, CHUNK)],
                        semg.at[b_a],
                    )

            return carry

        lax.fori_loop(0, n_phases + 2, step, 0)
        for b in range(NBUF):
            drain_out(b)

    run = pl.kernel(
        body,
        out_type=jax.ShapeDtypeStruct((d0, d1_size, EMBED_DIM), jnp.float32),
        mesh=mesh,
        scratch_types=[
            pltpu.VMEM((n_chunks, CHUNK), jnp.int32),
            pltpu.VMEM((n_chunks, CHUNK), jnp.int32),
            pltpu.VMEM((NBUF, PHASE_ROWS, EMBED_DIM), jnp.float32),
            pltpu.SemaphoreType.DMA((NBUF,)),
            pltpu.SemaphoreType.DMA((NBUF,)),
        ],
        compiler_params=pltpu.CompilerParams(use_tc_tiling_on_sc=False),
    )
    return run(table, xf, cf)


def kernel(x, constant, table):
    d0, d1 = x.shape
    xf = x.T.reshape(NW, d1, CHUNK).astype(jnp.int32)
    cf = constant.T.reshape(NW, d1, CHUNK).astype(jnp.int32)
    return _sc_embed_add(table, xf, cf, d0, d1)
